# baseline (speedup 1.0000x reference)
import functools

import jax
import jax.numpy as jnp
from jax import lax
from jax.experimental import pallas as pl
from jax.experimental.pallas import tpu as pltpu
from jax.experimental.pallas import tpu_sc as plsc


@functools.lru_cache(maxsize=None)
def _make_gather(n_rows, d, chunk):
    info = plsc.get_sparse_core_info()
    nc, ns = info.num_cores, info.num_subcores
    nw = nc * ns
    assert n_rows % nw == 0
    b_per_w = n_rows // nw
    assert b_per_w % chunk == 0
    n_chunks = b_per_w // chunk
    mesh = plsc.VectorSubcoreMesh(core_axis_name="c", subcore_axis_name="s")

    @functools.partial(
        pl.kernel,
        mesh=mesh,
        out_type=jax.ShapeDtypeStruct((n_rows, d), jnp.float32),
        scratch_types=[
            pltpu.VMEM((b_per_w,), jnp.int32),
            pltpu.VMEM((chunk, d), jnp.float32),
            pltpu.SemaphoreType.DMA,
        ],
        compiler_params=pltpu.CompilerParams(use_tc_tiling_on_sc=False),
    )
    def gather_kernel(idx_hbm, table_hbm, out_hbm, idx_v, rows_v, sem):
        wid = lax.axis_index("s") * nc + lax.axis_index("c")
        base = wid * b_per_w
        pltpu.sync_copy(idx_hbm.at[pl.ds(base, b_per_w)], idx_v)

        def body(i, carry):
            off = i * chunk
            pltpu.async_copy(
                table_hbm.at[idx_v.at[pl.ds(off, chunk)]], rows_v, sem
            ).wait()
            pltpu.sync_copy(rows_v, out_hbm.at[pl.ds(base + off, chunk)])
            return carry

        lax.fori_loop(0, n_chunks, body, 0)

    return gather_kernel


def kernel(indices, table):
    b, h = indices.shape
    _, d = table.shape
    idx_flat = indices.reshape(b * h).astype(jnp.int32)
    out = _make_gather(b * h, d, 1024)(idx_flat, table)
    return out.reshape(b, h, d)


# speedup vs baseline: 1.0071x; 1.0071x over previous
import functools

import jax
import jax.numpy as jnp
from jax import lax
from jax.experimental import pallas as pl
from jax.experimental.pallas import tpu as pltpu
from jax.experimental.pallas import tpu_sc as plsc

_NBUF = 4


@functools.lru_cache(maxsize=None)
def _make_gather(n_rows, d, chunk):
    info = plsc.get_sparse_core_info()
    nc, ns = info.num_cores, info.num_subcores
    nw = nc * ns
    assert n_rows % nw == 0
    b_per_w = n_rows // nw
    assert b_per_w % chunk == 0
    n_chunks = b_per_w // chunk
    assert n_chunks % _NBUF == 0 and n_chunks >= 2 * _NBUF
    n_groups = n_chunks // _NBUF
    mesh = plsc.VectorSubcoreMesh(core_axis_name="c", subcore_axis_name="s")

    @functools.partial(
        pl.kernel,
        mesh=mesh,
        out_type=jax.ShapeDtypeStruct((n_rows, d), jnp.float32),
        scratch_types=[
            pltpu.VMEM((b_per_w,), jnp.int32),
            pltpu.VMEM((_NBUF, chunk, d), jnp.float32),
            [pltpu.SemaphoreType.DMA] * _NBUF,
            [pltpu.SemaphoreType.DMA] * _NBUF,
        ],
        compiler_params=pltpu.CompilerParams(use_tc_tiling_on_sc=False),
    )
    def gather_kernel(idx_hbm, table_hbm, out_hbm, idx_v, rows_v, gsems, ssems):
        wid = lax.axis_index("s") * nc + lax.axis_index("c")
        base = wid * b_per_w
        pltpu.sync_copy(idx_hbm.at[pl.ds(base, b_per_w)], idx_v)

        def g_desc(i, b):
            return pltpu.make_async_copy(
                table_hbm.at[idx_v.at[pl.ds(i * chunk, chunk)]],
                rows_v.at[b],
                gsems[b],
            )

        def s_desc(i, b):
            return pltpu.make_async_copy(
                rows_v.at[b],
                out_hbm.at[pl.ds(base + i * chunk, chunk)],
                ssems[b],
            )

        def step(j, b, wait_store, start_gather):
            if wait_store:
                s_desc(j - 2, (b + 2) % _NBUF).wait()
            if start_gather:
                g_desc(j + 2, (b + 2) % _NBUF).start()
            g_desc(j, b).wait()
            s_desc(j, b).start()

        g_desc(0, 0).start()
        g_desc(1, 1).start()
        for b in range(_NBUF):
            step(b, b, wait_store=b >= 2, start_gather=True)

        def body(g, carry):
            for b in range(_NBUF):
                step(g * _NBUF + b, b, wait_store=True, start_gather=True)
            return carry

        lax.fori_loop(1, n_groups - 1, body, 0)

        for b in range(_NBUF):
            j = (n_groups - 1) * _NBUF + b
            step(j, b, wait_store=True, start_gather=j + 2 < n_chunks)
        s_desc(n_chunks - 2, (n_chunks - 2) % _NBUF).wait()
        s_desc(n_chunks - 1, (n_chunks - 1) % _NBUF).wait()

    return gather_kernel


def kernel(indices, table):
    b, h = indices.shape
    _, d = table.shape
    idx_flat = indices.reshape(b * h).astype(jnp.int32)
    out = _make_gather(b * h, d, 320)(idx_flat, table)
    return out.reshape(b, h, d)
